# Initial kernel scaffold; baseline (speedup 1.0000x reference)
import jax
import jax.numpy as jnp
from jax.experimental import pallas as pl


def kernel(x, batch):
    raise NotImplementedError("write your pallas kernel here")


# speedup vs baseline: 4.3538x; 4.3538x over previous
import jax
import jax.numpy as jnp
from jax import lax
from jax.experimental import pallas as pl
from jax.experimental.pallas import tpu as pltpu
from jax.experimental.pallas import tpu_sc as plsc

N = 100000
D = 128
G = 512
NC = 2
NS = 16
NW = NC * NS
CHUNK = 128
NFULL = N // CHUNK
TAIL = N - NFULL * CHUNK
CPW = -(-(NFULL + 1) // NW)
GROWS = G // NS


def _sc_body(x_hbm, batch_hbm, part_hbm, chunkbuf, idxbuf, obuf, tailbuf,
             tailidx, acc):
    c = lax.axis_index("c")
    s = lax.axis_index("s")
    w = s * NC + c

    @pl.loop(0, GROWS)
    def _zero_rows(i):
        @pl.loop(0, D // 16)
        def _zero_lanes(k):
            obuf[i, pl.ds(k * 16, 16)] = jnp.zeros((16,), jnp.float32)

    pltpu.sync_copy(obuf, acc.at[pl.ds(s * GROWS, GROWS)])
    plsc.subcore_barrier()

    @pl.loop(0, CPW)
    def _chunks(j):
        g = w * CPW + j

        @pl.when(g < NFULL)
        def _do():
            row0 = pl.multiple_of(g * CHUNK, CHUNK)
            pltpu.sync_copy(x_hbm.at[pl.ds(row0, CHUNK)], chunkbuf)
            pltpu.sync_copy(batch_hbm.at[pl.ds(row0, CHUNK)], idxbuf)
            pltpu.sync_copy(chunkbuf, acc.at[idxbuf], add=True)

    @pl.when(w == NW - 1)
    def _tail():
        row0 = pl.multiple_of(NFULL * CHUNK, CHUNK)
        pltpu.sync_copy(x_hbm.at[pl.ds(row0, TAIL)], tailbuf)
        pltpu.sync_copy(batch_hbm.at[pl.ds(row0, TAIL)], tailidx)
        pltpu.sync_copy(tailbuf, acc.at[tailidx], add=True)

    plsc.subcore_barrier()

    pltpu.sync_copy(acc.at[pl.ds(s * GROWS, GROWS)], obuf)
    pltpu.sync_copy(obuf, part_hbm.at[c, pl.ds(s * GROWS, GROWS)])


def _tc_add(p_ref, o_ref):
    o_ref[...] = p_ref[0] + p_ref[1]


@jax.jit
def _pool(x, batch):
    mesh = plsc.VectorSubcoreMesh(core_axis_name="c", subcore_axis_name="s",
                                  num_cores=NC, num_subcores=NS)
    partials = pl.kernel(
        _sc_body,
        out_type=jax.ShapeDtypeStruct((NC, G, D), jnp.float32),
        mesh=mesh,
        scratch_types=[
            pltpu.VMEM((CHUNK, D), jnp.float32),
            pltpu.VMEM((CHUNK,), jnp.int32),
            pltpu.VMEM((GROWS, D), jnp.float32),
            pltpu.VMEM((TAIL, D), jnp.float32),
            pltpu.VMEM((TAIL,), jnp.int32),
            pltpu.VMEM_SHARED((G, D), jnp.float32),
        ],
    )(x, batch)
    return pl.pallas_call(
        _tc_add,
        out_shape=jax.ShapeDtypeStruct((G, D), jnp.float32),
    )(partials)


def kernel(x, batch):
    return _pool(x, batch.astype(jnp.int32))


# speedup vs baseline: 6.8267x; 1.5680x over previous
import jax
import jax.numpy as jnp
from jax import lax
from jax.experimental import pallas as pl
from jax.experimental.pallas import tpu as pltpu
from jax.experimental.pallas import tpu_sc as plsc

N = 100000
D = 128
G = 512
NC = 2
NS = 16
NW = NC * NS
IDXW = 128
CHUNK = 256
NSLOT = N // CHUNK
TAIL = N - NSLOT * CHUNK
TAILBASE = NSLOT * CHUNK
TSTEPS = -(-NSLOT // NW) + 1
GROWS = G // NS


def _sc_body(x_hbm, batch_hbm, part_hbm, xb0, xb1, ib0, ib1, obuf, tbuf,
             tidxa, tidxb, acc, gs0, gs1, is0, is1):
    c = lax.axis_index("c")
    s = lax.axis_index("s")
    w = s * NC + c
    xbufs, ibufs = (xb0, xb1), (ib0, ib1)
    gsems, isems = (gs0, gs1), (is0, is1)

    @pl.loop(0, GROWS)
    def _zero_rows(i):
        @pl.loop(0, D // 16)
        def _zero_lanes(k):
            obuf[i, pl.ds(k * 16, 16)] = jnp.zeros((16,), jnp.float32)

    pltpu.sync_copy(obuf, acc.at[pl.ds(s * GROWS, GROWS)])
    plsc.subcore_barrier()

    def issue(slot, b):
        @pl.when(slot < NSLOT)
        def _():
            row0 = pl.multiple_of(slot * CHUNK, CHUNK)
            pltpu.async_copy(x_hbm.at[pl.ds(row0, CHUNK)], xbufs[b],
                             gsems[b])
            pltpu.async_copy(batch_hbm.at[pl.ds(row0, IDXW)],
                             ibufs[b].at[0], isems[b])
            pltpu.async_copy(batch_hbm.at[pl.ds(row0 + IDXW, IDXW)],
                             ibufs[b].at[1], isems[b])

    def process(slot, b):
        @pl.when(slot < NSLOT)
        def _():
            row0 = pl.multiple_of(slot * CHUNK, CHUNK)
            pltpu.make_async_copy(x_hbm.at[pl.ds(row0, CHUNK)], xbufs[b],
                                  gsems[b]).wait()
            pltpu.make_async_copy(batch_hbm.at[pl.ds(row0, IDXW)],
                                  ibufs[b].at[0], isems[b]).wait()
            pltpu.make_async_copy(batch_hbm.at[pl.ds(row0 + IDXW, IDXW)],
                                  ibufs[b].at[1], isems[b]).wait()
            pltpu.sync_copy(xbufs[b].at[pl.ds(0, IDXW)],
                            acc.at[ibufs[b].at[0]], add=True)
            pltpu.sync_copy(xbufs[b].at[pl.ds(IDXW, IDXW)],
                            acc.at[ibufs[b].at[1]], add=True)

    issue(w, 0)
    issue(w + NW, 1)

    @pl.loop(0, TSTEPS, step=2)
    def _main(t):
        s0 = w + NW * t
        process(s0, 0)
        issue(s0 + 2 * NW, 0)
        s1 = w + NW * (t + 1)
        process(s1, 1)
        issue(s1 + 2 * NW, 1)

    @pl.when(w == NW - 1)
    def _tail():
        row0 = pl.multiple_of(TAILBASE, CHUNK)
        pltpu.sync_copy(x_hbm.at[pl.ds(row0, TAIL)], tbuf)
        pltpu.sync_copy(batch_hbm.at[pl.ds(row0, IDXW)], tidxa)
        pltpu.sync_copy(batch_hbm.at[pl.ds(row0 + IDXW, TAIL - IDXW)], tidxb)
        pltpu.sync_copy(tbuf.at[pl.ds(0, IDXW)], acc.at[tidxa], add=True)
        pltpu.sync_copy(tbuf.at[pl.ds(IDXW, TAIL - IDXW)], acc.at[tidxb],
                        add=True)

    plsc.subcore_barrier()

    pltpu.sync_copy(acc.at[pl.ds(s * GROWS, GROWS)], obuf)
    pltpu.sync_copy(obuf, part_hbm.at[c, pl.ds(s * GROWS, GROWS)])


def _tc_add(p_ref, o_ref):
    o_ref[...] = p_ref[0] + p_ref[1]


@jax.jit
def _pool(x, batch):
    mesh = plsc.VectorSubcoreMesh(core_axis_name="c", subcore_axis_name="s",
                                  num_cores=NC, num_subcores=NS)
    partials = pl.kernel(
        _sc_body,
        out_type=jax.ShapeDtypeStruct((NC, G, D), jnp.float32),
        mesh=mesh,
        scratch_types=[
            pltpu.VMEM((CHUNK, D), jnp.float32),
            pltpu.VMEM((CHUNK, D), jnp.float32),
            pltpu.VMEM((2, IDXW), jnp.int32),
            pltpu.VMEM((2, IDXW), jnp.int32),
            pltpu.VMEM((GROWS, D), jnp.float32),
            pltpu.VMEM((TAIL, D), jnp.float32),
            pltpu.VMEM((IDXW,), jnp.int32),
            pltpu.VMEM((TAIL - IDXW,), jnp.int32),
            pltpu.VMEM_SHARED((G, D), jnp.float32),
            pltpu.SemaphoreType.DMA,
            pltpu.SemaphoreType.DMA,
            pltpu.SemaphoreType.DMA,
            pltpu.SemaphoreType.DMA,
        ],
    )(x, batch)
    return pl.pallas_call(
        _tc_add,
        out_shape=jax.ShapeDtypeStruct((G, D), jnp.float32),
    )(partials)


def kernel(x, batch):
    return _pool(x, batch.astype(jnp.int32))
